# baseline (speedup 1.0000x reference)
import dataclasses
import functools

import jax
import jax.numpy as jnp
from jax import lax
from jax.experimental import pallas as pl
from jax.experimental.pallas import tpu as pltpu
from jax.experimental.pallas import tpu_sc as plsc

DIM = 768
NUM_H = 64
NUM_W = 64
NC = 2
NS = 16
NW = NC * NS
LANES = 16
GROUP = 16
UNROLL = 8
SC_ROWS = 12288
TC_BLK = 512


def _sc_body(hi_hbm, wi_hbm, hemb_hbm, wemb_hbm, out_hbm,
             htab, wtab, obuf, hiv, wiv, sem_out, *, chunk, row0):
    c = lax.axis_index("c")
    s = lax.axis_index("s")
    wid = s * NC + c
    base = row0 + wid * chunk
    ngroups = chunk // GROUP

    pltpu.sync_copy(hemb_hbm, htab)
    pltpu.sync_copy(wemb_hbm, wtab)
    pltpu.sync_copy(hi_hbm.at[pl.ds(base, chunk)], hiv)
    pltpu.sync_copy(wi_hbm.at[pl.ds(base, chunk)], wiv)

    @pl.loop(0, ngroups, step=2)
    def _(g):
        for b in range(2):
            gg = g + b
            ob = obuf.at[pl.ds(b * GROUP * DIM, GROUP * DIM)]

            @pl.when(gg >= 2)
            def _():
                pltpu.make_async_copy(
                    ob, out_hbm.at[pl.ds(base * DIM, GROUP * DIM)],
                    sem_out).wait()

            hv = hiv[pl.ds(gg * GROUP, LANES)] * DIM
            wv = wiv[pl.ds(gg * GROUP, LANES)] * DIM

            for k in range(GROUP):
                hoff = pl.multiple_of(hv[k], 256)
                woff = pl.multiple_of(wv[k], 256)

                @plsc.parallel_loop(0, DIM, step=LANES * UNROLL)
                def _(cc):
                    for u in range(UNROLL):
                        col = cc + u * LANES
                        ob[pl.ds(k * DIM + col, LANES)] = (
                            htab[pl.ds(hoff + col, LANES)]
                            + wtab[pl.ds(woff + col, LANES)])

            pltpu.async_copy(
                ob,
                out_hbm.at[pl.ds((base + gg * GROUP) * DIM, GROUP * DIM)],
                sem_out)

    for b in range(2):
        pltpu.make_async_copy(
            obuf.at[pl.ds(b * GROUP * DIM, GROUP * DIM)],
            out_hbm.at[pl.ds(base * DIM, GROUP * DIM)],
            sem_out).wait()


def _sc_lookup(hi, wi, hemb_flat, wemb_flat, row0, nrows):
    n = hi.shape[0]
    chunk = nrows // NW

    mesh = plsc.VectorSubcoreMesh(core_axis_name="c", subcore_axis_name="s")
    cp = pltpu.CompilerParams()
    if "needs_layout_passes" in pltpu.CompilerParams.__dataclass_fields__:
        cp = dataclasses.replace(cp, needs_layout_passes=False)
    run = pl.kernel(
        functools.partial(_sc_body, chunk=chunk, row0=row0),
        out_type=jax.ShapeDtypeStruct((n * DIM,), jnp.float32),
        mesh=mesh,
        scratch_types=[
            pltpu.VMEM((NUM_H * DIM,), jnp.float32),
            pltpu.VMEM((NUM_W * DIM,), jnp.float32),
            pltpu.VMEM((2 * GROUP * DIM,), jnp.float32),
            pltpu.VMEM((chunk,), jnp.int32),
            pltpu.VMEM((chunk,), jnp.int32),
            pltpu.SemaphoreType.DMA,
        ],
        compiler_params=cp,
    )
    return run(hi, wi, hemb_flat, wemb_flat)


def _tc_body(hi_ref, wi_ref, thi_ref, tlo_ref, prev_ref, out_ref):
    idx_h = hi_ref[0, 0, :]
    idx_w = wi_ref[0, 0, :]
    ioh = jax.lax.broadcasted_iota(jnp.int32, (TC_BLK, NUM_H), 1)
    oh_h = (idx_h[:, None] == ioh).astype(jnp.bfloat16)
    oh_w = (idx_w[:, None] == ioh).astype(jnp.bfloat16)
    oh = jnp.concatenate([oh_h, oh_w], axis=1)
    acc = jnp.dot(oh, thi_ref[...], preferred_element_type=jnp.float32)
    acc += jnp.dot(oh, tlo_ref[...], preferred_element_type=jnp.float32)
    out_ref[...] = acc


def _tc_lookup_into(prev, hi, wi, t_hi, t_lo, nt):
    n = prev.shape[0]
    nb = nt // TC_BLK
    hi3 = hi.reshape(nb, 1, TC_BLK)
    wi3 = wi.reshape(nb, 1, TC_BLK)
    return pl.pallas_call(
        _tc_body,
        grid=(nb,),
        in_specs=[
            pl.BlockSpec((1, 1, TC_BLK), lambda i: (i, 0, 0)),
            pl.BlockSpec((1, 1, TC_BLK), lambda i: (i, 0, 0)),
            pl.BlockSpec((2 * NUM_H, DIM), lambda i: (0, 0)),
            pl.BlockSpec((2 * NUM_H, DIM), lambda i: (0, 0)),
            pl.BlockSpec(memory_space=pl.ANY),
        ],
        out_specs=pl.BlockSpec((TC_BLK, DIM), lambda i: (i, 0)),
        out_shape=jax.ShapeDtypeStruct((n, DIM), jnp.float32),
        input_output_aliases={4: 0},
    )(hi3, wi3, t_hi, t_lo, prev)


@jax.jit
def kernel(h_indices, w_indices, h_emb, w_emb):
    bsz, seq = h_indices.shape
    n = bsz * seq
    hi = h_indices.reshape(n).astype(jnp.int32)
    wi = w_indices.reshape(n).astype(jnp.int32)
    hemb_flat = h_emb.reshape(NUM_H * DIM)
    wemb_flat = w_emb.reshape(NUM_W * DIM)

    t_full = jnp.concatenate([h_emb, w_emb], axis=0)
    t_hi = t_full.astype(jnp.bfloat16)
    t_lo = (t_full - t_hi.astype(jnp.float32)).astype(jnp.bfloat16)

    nt = n - SC_ROWS
    out_sc = _sc_lookup(hi, wi, hemb_flat, wemb_flat, nt, SC_ROWS)
    out = _tc_lookup_into(out_sc.reshape(n, DIM), hi[:nt], wi[:nt],
                          t_hi, t_lo, nt)
    return out.reshape(bsz, seq, DIM)


# speedup vs baseline: 1.3809x; 1.3809x over previous
import dataclasses
import functools

import jax
import jax.numpy as jnp
from jax import lax
from jax.experimental import pallas as pl
from jax.experimental.pallas import tpu as pltpu
from jax.experimental.pallas import tpu_sc as plsc

DIM = 768
NUM_H = 64
NUM_W = 64
NC = 2
NS = 16
NW = NC * NS
LANES = 16
GROUP = 16
UNROLL = 8
SC_ROWS = 12288
TC_BLK = 512


def _sc_body(hi_hbm, wi_hbm, hemb_hbm, wemb_hbm, out_hbm,
             htab, wtab, obuf, hiv, wiv, sem_out, *, chunk, row0):
    c = lax.axis_index("c")
    s = lax.axis_index("s")
    wid = s * NC + c
    base = row0 + wid * chunk
    ngroups = chunk // GROUP

    pltpu.sync_copy(hemb_hbm, htab)
    pltpu.sync_copy(wemb_hbm, wtab)
    pltpu.sync_copy(hi_hbm.at[pl.ds(base, chunk)], hiv)
    pltpu.sync_copy(wi_hbm.at[pl.ds(base, chunk)], wiv)

    @pl.loop(0, ngroups, step=2)
    def _(g):
        for b in range(2):
            gg = g + b
            ob = obuf.at[pl.ds(b * GROUP * DIM, GROUP * DIM)]

            @pl.when(gg >= 2)
            def _():
                pltpu.make_async_copy(
                    ob, out_hbm.at[pl.ds(base * DIM, GROUP * DIM)],
                    sem_out).wait()

            hv = hiv[pl.ds(gg * GROUP, LANES)] * DIM
            wv = wiv[pl.ds(gg * GROUP, LANES)] * DIM

            for k in range(GROUP):
                hoff = pl.multiple_of(hv[k], 256)
                woff = pl.multiple_of(wv[k], 256)

                @plsc.parallel_loop(0, DIM, step=LANES * UNROLL)
                def _(cc):
                    for u in range(UNROLL):
                        col = cc + u * LANES
                        ob[pl.ds(k * DIM + col, LANES)] = (
                            htab[pl.ds(hoff + col, LANES)]
                            + wtab[pl.ds(woff + col, LANES)])

            pltpu.async_copy(
                ob,
                out_hbm.at[pl.ds((base + gg * GROUP) * DIM, GROUP * DIM)],
                sem_out)

    for b in range(2):
        pltpu.make_async_copy(
            obuf.at[pl.ds(b * GROUP * DIM, GROUP * DIM)],
            out_hbm.at[pl.ds(base * DIM, GROUP * DIM)],
            sem_out).wait()


def _sc_lookup(hi, wi, hemb_flat, wemb_flat, row0, nrows):
    n = hi.shape[0]
    chunk = nrows // NW

    mesh = plsc.VectorSubcoreMesh(core_axis_name="c", subcore_axis_name="s")
    cp = pltpu.CompilerParams()
    if "needs_layout_passes" in pltpu.CompilerParams.__dataclass_fields__:
        cp = dataclasses.replace(cp, needs_layout_passes=False)
    run = pl.kernel(
        functools.partial(_sc_body, chunk=chunk, row0=row0),
        out_type=jax.ShapeDtypeStruct((n * DIM,), jnp.float32),
        mesh=mesh,
        scratch_types=[
            pltpu.VMEM((NUM_H * DIM,), jnp.float32),
            pltpu.VMEM((NUM_W * DIM,), jnp.float32),
            pltpu.VMEM((2 * GROUP * DIM,), jnp.float32),
            pltpu.VMEM((chunk,), jnp.int32),
            pltpu.VMEM((chunk,), jnp.int32),
            pltpu.SemaphoreType.DMA,
        ],
        compiler_params=cp,
        cost_estimate=pl.CostEstimate(
            flops=int(nrows) * DIM,
            bytes_accessed=int(nrows) * DIM * 12,
            transcendentals=0,
        ),
    )
    return run(hi, wi, hemb_flat, wemb_flat)


def _tc_body(hi_ref, wi_ref, thi_ref, tlo_ref, out_ref):
    idx_h = hi_ref[0, 0, :]
    idx_w = wi_ref[0, 0, :]
    ioh = jax.lax.broadcasted_iota(jnp.int32, (TC_BLK, NUM_H), 1)
    oh_h = (idx_h[:, None] == ioh).astype(jnp.bfloat16)
    oh_w = (idx_w[:, None] == ioh).astype(jnp.bfloat16)
    oh = jnp.concatenate([oh_h, oh_w], axis=1)
    acc = jnp.dot(oh, thi_ref[...], preferred_element_type=jnp.float32)
    acc += jnp.dot(oh, tlo_ref[...], preferred_element_type=jnp.float32)
    out_ref[...] = acc


def _tc_lookup(hi, wi, t_hi, t_lo):
    n = hi.shape[0]
    nb = n // TC_BLK
    hi3 = hi.reshape(nb, 1, TC_BLK)
    wi3 = wi.reshape(nb, 1, TC_BLK)
    return pl.pallas_call(
        _tc_body,
        grid=(nb,),
        in_specs=[
            pl.BlockSpec((1, 1, TC_BLK), lambda i: (i, 0, 0)),
            pl.BlockSpec((1, 1, TC_BLK), lambda i: (i, 0, 0)),
            pl.BlockSpec((2 * NUM_H, DIM), lambda i: (0, 0)),
            pl.BlockSpec((2 * NUM_H, DIM), lambda i: (0, 0)),
        ],
        out_specs=pl.BlockSpec((TC_BLK, DIM), lambda i: (i, 0)),
        out_shape=jax.ShapeDtypeStruct((n, DIM), jnp.float32),
    )(hi3, wi3, t_hi, t_lo)


@jax.jit
def kernel(h_indices, w_indices, h_emb, w_emb):
    bsz, seq = h_indices.shape
    n = bsz * seq
    hi = h_indices.reshape(n).astype(jnp.int32)
    wi = w_indices.reshape(n).astype(jnp.int32)
    hemb_flat = h_emb.reshape(NUM_H * DIM)
    wemb_flat = w_emb.reshape(NUM_W * DIM)

    t_full = jnp.concatenate([h_emb, w_emb], axis=0)
    t_hi = t_full.astype(jnp.bfloat16)
    t_lo = (t_full - t_hi.astype(jnp.float32)).astype(jnp.bfloat16)

    nt = n - SC_ROWS
    out_sc = _sc_lookup(hi[nt:], wi[nt:], hemb_flat, wemb_flat, 0, SC_ROWS)
    out_tc = _tc_lookup(hi[:nt], wi[:nt], t_hi, t_lo)
    out = jnp.concatenate([out_tc, out_sc.reshape(SC_ROWS, DIM)], axis=0)
    return out.reshape(bsz, seq, DIM)


# speedup vs baseline: 1.4535x; 1.0526x over previous
import dataclasses
import functools

import jax
import jax.numpy as jnp
from jax import lax
from jax.experimental import pallas as pl
from jax.experimental.pallas import tpu as pltpu
from jax.experimental.pallas import tpu_sc as plsc

DIM = 768
NUM_H = 64
NUM_W = 64
NC = 2
NS = 16
NW = NC * NS
LANES = 16
GROUP = 16
UNROLL = 8
SC_ROWS = 12288
TC_BLK = 512


def _sc_body(hi_hbm, wi_hbm, hemb_hbm, wemb_hbm, out_hbm,
             htab, wtab, obuf, hiv, wiv, sem_out, *, chunk, row0):
    c = lax.axis_index("c")
    s = lax.axis_index("s")
    wid = s * NC + c
    base = row0 + wid * chunk
    ngroups = chunk // GROUP

    pltpu.sync_copy(hemb_hbm, htab)
    pltpu.sync_copy(wemb_hbm, wtab)
    pltpu.sync_copy(hi_hbm.at[pl.ds(base, chunk)], hiv)
    pltpu.sync_copy(wi_hbm.at[pl.ds(base, chunk)], wiv)

    @pl.loop(0, ngroups, step=2)
    def _(g):
        for b in range(2):
            gg = g + b
            ob = obuf.at[pl.ds(b * GROUP * DIM, GROUP * DIM)]

            @pl.when(gg >= 2)
            def _():
                pltpu.make_async_copy(
                    ob, out_hbm.at[pl.ds(base * DIM, GROUP * DIM)],
                    sem_out).wait()

            hv = hiv[pl.ds(gg * GROUP, LANES)] * DIM
            wv = wiv[pl.ds(gg * GROUP, LANES)] * DIM

            for k in range(GROUP):
                hoff = pl.multiple_of(hv[k], 256)
                woff = pl.multiple_of(wv[k], 256)

                @plsc.parallel_loop(0, DIM, step=LANES * UNROLL)
                def _(cc):
                    for u in range(UNROLL):
                        col = cc + u * LANES
                        ob[pl.ds(k * DIM + col, LANES)] = (
                            htab[pl.ds(hoff + col, LANES)]
                            + wtab[pl.ds(woff + col, LANES)])

            pltpu.async_copy(
                ob,
                out_hbm.at[pl.ds((base + gg * GROUP) * DIM, GROUP * DIM)],
                sem_out)

    for b in range(2):
        pltpu.make_async_copy(
            obuf.at[pl.ds(b * GROUP * DIM, GROUP * DIM)],
            out_hbm.at[pl.ds(base * DIM, GROUP * DIM)],
            sem_out).wait()


def _sc_lookup(hi, wi, hemb_flat, wemb_flat, row0, nrows):
    n = hi.shape[0]
    chunk = nrows // NW

    mesh = plsc.VectorSubcoreMesh(core_axis_name="c", subcore_axis_name="s")
    cp = pltpu.CompilerParams()
    if "needs_layout_passes" in pltpu.CompilerParams.__dataclass_fields__:
        cp = dataclasses.replace(cp, needs_layout_passes=False)
    run = pl.kernel(
        functools.partial(_sc_body, chunk=chunk, row0=row0),
        out_type=jax.ShapeDtypeStruct((n * DIM,), jnp.float32),
        mesh=mesh,
        scratch_types=[
            pltpu.VMEM((NUM_H * DIM,), jnp.float32),
            pltpu.VMEM((NUM_W * DIM,), jnp.float32),
            pltpu.VMEM((2 * GROUP * DIM,), jnp.float32),
            pltpu.VMEM((chunk,), jnp.int32),
            pltpu.VMEM((chunk,), jnp.int32),
            pltpu.SemaphoreType.DMA,
        ],
        compiler_params=cp,
        cost_estimate=pl.CostEstimate(
            flops=int(nrows) * DIM,
            bytes_accessed=int(nrows) * DIM * 12,
            transcendentals=0,
        ),
    )
    return run(hi, wi, hemb_flat, wemb_flat)


def _tc_body(nbt, hi_ref, wi_ref, thi_ref, tlo_ref, sc_ref, out_ref):
    pid = pl.program_id(0)

    @pl.when(pid < nbt)
    def _():
        idx_h = hi_ref[0, 0, :]
        idx_w = wi_ref[0, 0, :]
        ioh = jax.lax.broadcasted_iota(jnp.int32, (TC_BLK, NUM_H), 1)
        oh_h = (idx_h[:, None] == ioh).astype(jnp.bfloat16)
        oh_w = (idx_w[:, None] == ioh).astype(jnp.bfloat16)
        oh = jnp.concatenate([oh_h, oh_w], axis=1)
        acc = jnp.dot(oh, thi_ref[...], preferred_element_type=jnp.float32)
        acc += jnp.dot(oh, tlo_ref[...], preferred_element_type=jnp.float32)
        out_ref[...] = acc

    @pl.when(pid >= nbt)
    def _():
        out_ref[...] = sc_ref[...]


def _tc_lookup(hi, wi, t_hi, t_lo, sc_out, nt):
    n = hi.shape[0]
    nb = n // TC_BLK
    nbt = nt // TC_BLK
    hi3 = hi.reshape(nb, 1, TC_BLK)
    wi3 = wi.reshape(nb, 1, TC_BLK)
    return pl.pallas_call(
        functools.partial(_tc_body, nbt),
        grid=(nb,),
        in_specs=[
            pl.BlockSpec((1, 1, TC_BLK), lambda i: (i, 0, 0)),
            pl.BlockSpec((1, 1, TC_BLK), lambda i: (i, 0, 0)),
            pl.BlockSpec((2 * NUM_H, DIM), lambda i: (0, 0)),
            pl.BlockSpec((2 * NUM_H, DIM), lambda i: (0, 0)),
            pl.BlockSpec((TC_BLK, DIM),
                         lambda i: (jnp.maximum(i - (n - SC_ROWS) // TC_BLK, 0), 0)),
        ],
        out_specs=pl.BlockSpec((TC_BLK, DIM), lambda i: (i, 0)),
        out_shape=jax.ShapeDtypeStruct((n, DIM), jnp.float32),
    )(hi3, wi3, t_hi, t_lo, sc_out)


@jax.jit
def kernel(h_indices, w_indices, h_emb, w_emb):
    bsz, seq = h_indices.shape
    n = bsz * seq
    hi = h_indices.reshape(n).astype(jnp.int32)
    wi = w_indices.reshape(n).astype(jnp.int32)
    hemb_flat = h_emb.reshape(NUM_H * DIM)
    wemb_flat = w_emb.reshape(NUM_W * DIM)

    t_full = jnp.concatenate([h_emb, w_emb], axis=0)
    t_hi = t_full.astype(jnp.bfloat16)
    t_lo = (t_full - t_hi.astype(jnp.float32)).astype(jnp.bfloat16)

    nt = n - SC_ROWS
    out_sc = _sc_lookup(hi[nt:], wi[nt:], hemb_flat, wemb_flat, 0, SC_ROWS)
    out = _tc_lookup(hi, wi, t_hi, t_lo, out_sc.reshape(SC_ROWS, DIM), nt)
    return out.reshape(bsz, seq, DIM)


# speedup vs baseline: 1.6812x; 1.1567x over previous
import dataclasses
import functools

import jax
import jax.numpy as jnp
from jax import lax
from jax.experimental import pallas as pl
from jax.experimental.pallas import tpu as pltpu
from jax.experimental.pallas import tpu_sc as plsc

DIM = 768
NUM_H = 64
NUM_W = 64
NC = 2
NS = 16
NW = NC * NS
LANES = 16
GROUP = 16
UNROLL = 8
SC_ROWS = 8192
TC_BLK = 512


def _sc_body(hi_hbm, wi_hbm, hemb_hbm, wemb_hbm, out_hbm,
             htab, wtab, obuf, hiv, wiv, sem_out, *, chunk, row0):
    c = lax.axis_index("c")
    s = lax.axis_index("s")
    wid = s * NC + c
    base = row0 + wid * chunk
    ngroups = chunk // GROUP

    pltpu.sync_copy(hemb_hbm, htab)
    pltpu.sync_copy(wemb_hbm, wtab)
    pltpu.sync_copy(hi_hbm.at[pl.ds(base, chunk)], hiv)
    pltpu.sync_copy(wi_hbm.at[pl.ds(base, chunk)], wiv)

    @pl.loop(0, ngroups, step=2)
    def _(g):
        for b in range(2):
            gg = g + b
            ob = obuf.at[pl.ds(b * GROUP * DIM, GROUP * DIM)]

            @pl.when(gg >= 2)
            def _():
                pltpu.make_async_copy(
                    ob, out_hbm.at[pl.ds(base * DIM, GROUP * DIM)],
                    sem_out).wait()

            hv = hiv[pl.ds(gg * GROUP, LANES)] * DIM
            wv = wiv[pl.ds(gg * GROUP, LANES)] * DIM

            for k in range(GROUP):
                hoff = pl.multiple_of(hv[k], 256)
                woff = pl.multiple_of(wv[k], 256)

                @plsc.parallel_loop(0, DIM, step=LANES * UNROLL)
                def _(cc):
                    for u in range(UNROLL):
                        col = cc + u * LANES
                        ob[pl.ds(k * DIM + col, LANES)] = (
                            htab[pl.ds(hoff + col, LANES)]
                            + wtab[pl.ds(woff + col, LANES)])

            pltpu.async_copy(
                ob,
                out_hbm.at[pl.ds((base + gg * GROUP) * DIM, GROUP * DIM)],
                sem_out)

    for b in range(2):
        pltpu.make_async_copy(
            obuf.at[pl.ds(b * GROUP * DIM, GROUP * DIM)],
            out_hbm.at[pl.ds(base * DIM, GROUP * DIM)],
            sem_out).wait()


def _sc_lookup(hi, wi, hemb_flat, wemb_flat, row0, nrows):
    n = hi.shape[0]
    chunk = nrows // NW

    mesh = plsc.VectorSubcoreMesh(core_axis_name="c", subcore_axis_name="s")
    cp = pltpu.CompilerParams()
    if "needs_layout_passes" in pltpu.CompilerParams.__dataclass_fields__:
        cp = dataclasses.replace(cp, needs_layout_passes=False)
    run = pl.kernel(
        functools.partial(_sc_body, chunk=chunk, row0=row0),
        out_type=jax.ShapeDtypeStruct((n * DIM,), jnp.float32),
        mesh=mesh,
        scratch_types=[
            pltpu.VMEM((NUM_H * DIM,), jnp.float32),
            pltpu.VMEM((NUM_W * DIM,), jnp.float32),
            pltpu.VMEM((2 * GROUP * DIM,), jnp.float32),
            pltpu.VMEM((chunk,), jnp.int32),
            pltpu.VMEM((chunk,), jnp.int32),
            pltpu.SemaphoreType.DMA,
        ],
        compiler_params=cp,
        cost_estimate=pl.CostEstimate(
            flops=int(nrows) * DIM,
            bytes_accessed=int(nrows) * DIM * 12,
            transcendentals=0,
        ),
    )
    return run(hi, wi, hemb_flat, wemb_flat)


def _tc_body(nbt, hi_ref, wi_ref, thi_ref, tlo_ref, sc_ref, out_ref):
    pid = pl.program_id(0)

    @pl.when(pid < nbt)
    def _():
        idx_h = hi_ref[0, 0, :]
        idx_w = wi_ref[0, 0, :]
        ioh = jax.lax.broadcasted_iota(jnp.int32, (TC_BLK, NUM_H), 1)
        oh_h = (idx_h[:, None] == ioh).astype(jnp.bfloat16)
        oh_w = (idx_w[:, None] == ioh).astype(jnp.bfloat16)
        oh = jnp.concatenate([oh_h, oh_w], axis=1)
        acc = jnp.dot(oh, thi_ref[...], preferred_element_type=jnp.float32)
        acc += jnp.dot(oh, tlo_ref[...], preferred_element_type=jnp.float32)
        out_ref[...] = acc

    @pl.when(pid >= nbt)
    def _():
        out_ref[...] = sc_ref[...]


def _tc_lookup(hi, wi, t_hi, t_lo, sc_out, nt):
    n = hi.shape[0]
    nb = n // TC_BLK
    nbt = nt // TC_BLK
    hi3 = hi.reshape(nb, 1, TC_BLK)
    wi3 = wi.reshape(nb, 1, TC_BLK)
    return pl.pallas_call(
        functools.partial(_tc_body, nbt),
        grid=(nb,),
        in_specs=[
            pl.BlockSpec((1, 1, TC_BLK), lambda i: (i, 0, 0)),
            pl.BlockSpec((1, 1, TC_BLK), lambda i: (i, 0, 0)),
            pl.BlockSpec((2 * NUM_H, DIM), lambda i: (0, 0)),
            pl.BlockSpec((2 * NUM_H, DIM), lambda i: (0, 0)),
            pl.BlockSpec((TC_BLK, DIM),
                         lambda i: (jnp.maximum(i - (n - SC_ROWS) // TC_BLK, 0), 0)),
        ],
        out_specs=pl.BlockSpec((TC_BLK, DIM), lambda i: (i, 0)),
        out_shape=jax.ShapeDtypeStruct((n, DIM), jnp.float32),
    )(hi3, wi3, t_hi, t_lo, sc_out)


@jax.jit
def kernel(h_indices, w_indices, h_emb, w_emb):
    bsz, seq = h_indices.shape
    n = bsz * seq
    hi = h_indices.reshape(n).astype(jnp.int32)
    wi = w_indices.reshape(n).astype(jnp.int32)
    hemb_flat = h_emb.reshape(NUM_H * DIM)
    wemb_flat = w_emb.reshape(NUM_W * DIM)

    t_full = jnp.concatenate([h_emb, w_emb], axis=0)
    t_hi = t_full.astype(jnp.bfloat16)
    t_lo = (t_full - t_hi.astype(jnp.float32)).astype(jnp.bfloat16)

    nt = n - SC_ROWS
    out_sc = _sc_lookup(hi[nt:], wi[nt:], hemb_flat, wemb_flat, 0, SC_ROWS)
    out = _tc_lookup(hi, wi, t_hi, t_lo, out_sc.reshape(SC_ROWS, DIM), nt)
    return out.reshape(bsz, seq, DIM)
